# baseline (speedup 1.0000x reference)
import functools

import jax
import jax.numpy as jnp
from jax import lax
from jax.experimental import pallas as pl
from jax.experimental.pallas import tpu as pltpu
from jax.experimental.pallas import tpu_sc as plsc

EMBED = 768
FFN_D = 3072
NE = 8
NTOK = 2048
TOPK = 2
NA = NTOK * TOPK
TILE = 128
TSHIFT = 7
GMAX = 40
SLOTS = GMAX * TILE
NW = 32
APW = NA // NW


def _router_body(x_ref, wrt_ref, idx_ref, w_ref, loss_ref):
    x = x_ref[...]
    wrt = wrt_ref[...]
    logits = jnp.dot(x, wrt, preferred_element_type=jnp.float32)
    ids = jax.lax.broadcasted_iota(jnp.int32, logits.shape, 1)
    m1 = jnp.max(logits, axis=1, keepdims=True)
    i1 = jnp.min(jnp.where(logits == m1, ids, NE), axis=1, keepdims=True)
    l2 = jnp.where(ids == i1, -jnp.inf, logits)
    m2 = jnp.max(l2, axis=1, keepdims=True)
    i2 = jnp.min(jnp.where(l2 == m2, ids, NE), axis=1, keepdims=True)
    t = jnp.exp(m2 - m1)
    w1 = 1.0 / (1.0 + t)
    w2 = 1.0 - w1
    idx_ref[...] = jnp.concatenate([i1, i2], axis=1)
    w_ref[...] = jnp.concatenate([w1, w2], axis=1)
    ex = jnp.exp(logits - m1)
    probs = ex / jnp.sum(ex, axis=1, keepdims=True)
    usage = jnp.sum(probs, axis=0, keepdims=True) * (1.0 / NTOK)
    loss_ref[...] = NE * jnp.sum(usage * usage, axis=1, keepdims=True)


def _router(x_flat, Wr):
    return pl.pallas_call(
        _router_body,
        out_shape=(
            jax.ShapeDtypeStruct((NTOK, 2), jnp.int32),
            jax.ShapeDtypeStruct((NTOK, 2), jnp.float32),
            jax.ShapeDtypeStruct((1, 1), jnp.float32),
        ),
    )(x_flat, Wr.T)


def _dispatch(e_flat, x_flat):
    mesh = plsc.VectorSubcoreMesh(core_axis_name="c", subcore_axis_name="s", num_cores=2, num_subcores=16)

    @functools.partial(
        pl.kernel,
        out_type=(
            jax.ShapeDtypeStruct((NA,), jnp.int32),
            jax.ShapeDtypeStruct((SLOTS, EMBED), jnp.float32),
            jax.ShapeDtypeStruct((64,), jnp.int32),
        ),
        mesh=mesh,
        compiler_params=pltpu.CompilerParams(needs_layout_passes=False),
        scratch_types=[
            pltpu.VMEM((NA,), jnp.int32),
            pltpu.VMEM((APW,), jnp.int32),
            pltpu.VMEM((APW,), jnp.int32),
            pltpu.VMEM((APW, EMBED), jnp.float32),
            pltpu.VMEM((64,), jnp.int32),
            pltpu.SemaphoreType.DMA,
        ],
    )
    def k(e_hbm, x_hbm, pos_hbm, xs_hbm, meta_hbm,
          evmem, posbuf, tokbuf, rows, metabuf, sem):
        nc = 2
        wid = lax.axis_index("s") * nc + lax.axis_index("c")
        lanes = lax.iota(jnp.int32, 16)

        pltpu.sync_copy(e_hbm, evmem)
        myv0 = wid * (APW // 16)
        zero16 = jnp.zeros((16,), jnp.int32)

        def hbody(v, carry):
            hist_c, prefix_c = carry
            snap = jnp.broadcast_to(v == myv0, (16,))
            prefix_c = jnp.where(snap, hist_c, prefix_c)
            vec = evmem[pl.ds(v * 16, 16)]
            for e in range(NE):
                pc = plsc.all_reduce_population_count(vec == e)
                hist_c = hist_c + jnp.where(lanes == e, pc, 0)
            return hist_c, prefix_c

        totals, prefix_v = lax.fori_loop(0, NA // 16, hbody, (zero16, zero16))

        padded = (totals + (TILE - 1)) >> TSHIFT
        incl = plsc.cumsum(padded)
        tile_start = incl - padded
        start_vec = tile_start * TILE + prefix_v

        base_a = wid * APW
        for v8 in range(APW // 16):
            vec = evmem[pl.ds(base_a + v8 * 16, 16)]
            pos = jnp.zeros((16,), jnp.int32)
            for e in range(NE):
                m = vec == e
                mi = m.astype(jnp.int32)
                cm = plsc.cumsum(mi)
                cnt = jnp.sum(mi)
                s_e = jnp.sum(jnp.where(lanes == e, start_vec, 0))
                pos = pos + jnp.where(m, s_e + cm - 1, 0)
                start_vec = start_vec + jnp.where(lanes == e, cnt, 0)
            posbuf[pl.ds(v8 * 16, 16)] = pos
            tokbuf[pl.ds(v8 * 16, 16)] = (base_a + v8 * 16 + lanes) >> 1

        pltpu.sync_copy(posbuf, pos_hbm.at[pl.ds(base_a, APW)])
        pltpu.async_copy(x_hbm.at[tokbuf], rows, sem).wait()
        pltpu.async_copy(rows, xs_hbm.at[posbuf], sem).wait()

        @pl.when(wid == 0)
        def _():
            ntiles = jnp.sum(jnp.where(lanes == NE - 1, incl, 0))
            for gv in range(3):
                g_vec = gv * 16 + lanes
                te = jnp.full((16,), -1, jnp.int32)
                for e in range(NE):
                    s_e = jnp.sum(jnp.where(lanes == e, tile_start, 0))
                    te = te + (g_vec >= s_e).astype(jnp.int32)
                metabuf[pl.ds(gv * 16, 16)] = te
            metabuf[pl.ds(48, 16)] = jnp.full((16,), ntiles, jnp.int32)
            pltpu.sync_copy(metabuf, meta_hbm)

    return k(e_flat, x_flat)


def _ffn_body(meta_ref, xs_ref, wg_ref, wu_ref, wd_ref, ys_ref):
    g = pl.program_id(0)
    ntiles = meta_ref[48]

    @pl.when(g < ntiles)
    def _():
        xt = xs_ref[...].astype(jnp.bfloat16)
        gg = jnp.dot(xt, wg_ref[0], preferred_element_type=jnp.float32)
        s = gg * (1.0 / (1.0 + jnp.exp(-gg)))
        u = jnp.dot(xt, wu_ref[0], preferred_element_type=jnp.float32)
        h = (s * u).astype(jnp.bfloat16)
        ys_ref[...] = jnp.dot(h, wd_ref[0], preferred_element_type=jnp.float32)


def _ffn(meta, xs, Wg, Wu, Wd):
    grid_spec = pltpu.PrefetchScalarGridSpec(
        num_scalar_prefetch=1,
        grid=(GMAX,),
        in_specs=[
            pl.BlockSpec((TILE, EMBED), lambda g, te: (g, 0)),
            pl.BlockSpec((1, EMBED, FFN_D), lambda g, te: (te[g], 0, 0),
                         pipeline_mode=pl.Buffered(buffer_count=2)),
            pl.BlockSpec((1, EMBED, FFN_D), lambda g, te: (te[g], 0, 0),
                         pipeline_mode=pl.Buffered(buffer_count=2)),
            pl.BlockSpec((1, FFN_D, EMBED), lambda g, te: (te[g], 0, 0),
                         pipeline_mode=pl.Buffered(buffer_count=2)),
        ],
        out_specs=pl.BlockSpec((TILE, EMBED), lambda g, te: (g, 0)),
    )
    return pl.pallas_call(
        _ffn_body,
        grid_spec=grid_spec,
        out_shape=jax.ShapeDtypeStruct((SLOTS, EMBED), jnp.float32),
    )(meta, xs, Wg, Wu, Wd)


def _gather_back(ys, pos):
    mesh = plsc.VectorSubcoreMesh(core_axis_name="c", subcore_axis_name="s", num_cores=2, num_subcores=16)

    @functools.partial(
        pl.kernel,
        out_type=jax.ShapeDtypeStruct((NA, EMBED), jnp.float32),
        mesh=mesh,
        compiler_params=pltpu.CompilerParams(needs_layout_passes=False),
        scratch_types=[
            pltpu.VMEM((APW,), jnp.int32),
            pltpu.VMEM((APW,), jnp.int32),
            pltpu.VMEM((APW, EMBED), jnp.float32),
            pltpu.SemaphoreType.DMA,
        ],
    )
    def k(ys_hbm, pos_hbm, yg_hbm, posv, dstv, rows, sem):
        nc = 2
        wid = lax.axis_index("s") * nc + lax.axis_index("c")
        base = wid * APW
        lanes = lax.iota(jnp.int32, 16)
        pltpu.sync_copy(pos_hbm.at[pl.ds(base, APW)], posv)
        for v8 in range(APW // 16):
            a = base + v8 * 16 + lanes
            dstv[pl.ds(v8 * 16, 16)] = (a & 1) * NTOK + (a >> 1)
        pltpu.async_copy(ys_hbm.at[posv], rows, sem).wait()
        pltpu.async_copy(rows, yg_hbm.at[dstv], sem).wait()

    return k(ys, pos)


def _combine_body(w_ref, ya_ref, yb_ref, out_ref):
    m = pl.program_id(0)
    wm = w_ref[pl.ds(m * 256, 256), :]
    out_ref[...] = ya_ref[...] * wm[:, 0:1] + yb_ref[...] * wm[:, 1:2]


def _combine(w, yg):
    nblk = NTOK // 256
    return pl.pallas_call(
        _combine_body,
        grid=(nblk,),
        in_specs=[
            pl.BlockSpec((NTOK, 2), lambda m: (0, 0)),
            pl.BlockSpec((256, EMBED), lambda m: (m, 0)),
            pl.BlockSpec((256, EMBED), lambda m: (m + nblk, 0)),
        ],
        out_specs=pl.BlockSpec((256, EMBED), lambda m: (m, 0)),
        out_shape=jax.ShapeDtypeStruct((NTOK, EMBED), jnp.float32),
    )(w, yg, yg)


def kernel(x, Wr, Wg, Wu, Wd):
    B, T, D = x.shape
    x_flat = x.reshape(B * T, D)
    idx, w, loss = _router(x_flat, Wr)
    e_flat = idx.reshape(NA)
    pos, xs, meta = _dispatch(e_flat, x_flat)
    ys = _ffn(meta, xs, Wg.astype(jnp.bfloat16), Wu.astype(jnp.bfloat16),
              Wd.astype(jnp.bfloat16))
    yg = _gather_back(ys, pos)
    out_flat = _combine(w, yg)
    return out_flat.reshape(B, T, D), loss.reshape(())


# speedup vs baseline: 1.3162x; 1.3162x over previous
import functools

import jax
import jax.numpy as jnp
from jax import lax
from jax.experimental import pallas as pl
from jax.experimental.pallas import tpu as pltpu
from jax.experimental.pallas import tpu_sc as plsc

EMBED = 768
FFN_D = 3072
NE = 8
NTOK = 2048
TOPK = 2
NA = NTOK * TOPK
TILE = 128
TSHIFT = 7
GMAX = 40
SLOTS = GMAX * TILE
NW = 32
APW = NA // NW


def _router_body(x_ref, wrt_ref, idx_ref, w_ref, loss_ref):
    x = x_ref[...]
    wrt = wrt_ref[...]
    logits = jnp.dot(x, wrt, preferred_element_type=jnp.float32)
    ids = jax.lax.broadcasted_iota(jnp.int32, logits.shape, 1)
    m1 = jnp.max(logits, axis=1, keepdims=True)
    i1 = jnp.min(jnp.where(logits == m1, ids, NE), axis=1, keepdims=True)
    l2 = jnp.where(ids == i1, -jnp.inf, logits)
    m2 = jnp.max(l2, axis=1, keepdims=True)
    i2 = jnp.min(jnp.where(l2 == m2, ids, NE), axis=1, keepdims=True)
    t = jnp.exp(m2 - m1)
    w1 = 1.0 / (1.0 + t)
    w2 = 1.0 - w1
    idx_ref[...] = jnp.concatenate([i1, i2], axis=1)
    w_ref[...] = jnp.concatenate([w1, w2], axis=1)
    ex = jnp.exp(logits - m1)
    probs = ex / jnp.sum(ex, axis=1, keepdims=True)
    usage = jnp.sum(probs, axis=0, keepdims=True) * (1.0 / NTOK)
    loss_ref[...] = NE * jnp.sum(usage * usage, axis=1, keepdims=True)


def _router(x_flat, Wr):
    return pl.pallas_call(
        _router_body,
        out_shape=(
            jax.ShapeDtypeStruct((NTOK, 2), jnp.int32),
            jax.ShapeDtypeStruct((NTOK, 2), jnp.float32),
            jax.ShapeDtypeStruct((1, 1), jnp.float32),
        ),
    )(x_flat, Wr.T)


def _dispatch(e_flat, x_flat):
    mesh = plsc.VectorSubcoreMesh(core_axis_name="c", subcore_axis_name="s", num_cores=2, num_subcores=16)

    @functools.partial(
        pl.kernel,
        out_type=(
            jax.ShapeDtypeStruct((NA,), jnp.int32),
            jax.ShapeDtypeStruct((SLOTS, EMBED), jnp.float32),
            jax.ShapeDtypeStruct((64,), jnp.int32),
        ),
        mesh=mesh,
        compiler_params=pltpu.CompilerParams(needs_layout_passes=False),
        scratch_types=[
            pltpu.VMEM((NA,), jnp.int32),
            pltpu.VMEM((APW,), jnp.int32),
            pltpu.VMEM((APW,), jnp.int32),
            pltpu.VMEM((APW, EMBED), jnp.float32),
            pltpu.VMEM((64,), jnp.int32),
            pltpu.SemaphoreType.DMA,
        ],
    )
    def k(e_hbm, x_hbm, pos_hbm, xs_hbm, meta_hbm,
          evmem, posbuf, tokbuf, rows, metabuf, sem):
        nc = 2
        wid = lax.axis_index("s") * nc + lax.axis_index("c")
        lanes = lax.iota(jnp.int32, 16)

        pltpu.sync_copy(e_hbm, evmem)
        myv0 = wid * (APW // 16)
        zero16 = jnp.zeros((16,), jnp.int32)

        def hbody(v, carry):
            hist_c, prefix_c = carry
            snap = jnp.broadcast_to(v == myv0, (16,))
            prefix_c = jnp.where(snap, hist_c, prefix_c)
            vec = evmem[pl.ds(v * 16, 16)]
            for e in range(NE):
                pc = plsc.all_reduce_population_count(vec == e)
                hist_c = hist_c + jnp.where(lanes == e, pc, 0)
            return hist_c, prefix_c

        totals, prefix_v = lax.fori_loop(0, NA // 16, hbody, (zero16, zero16))

        padded = (totals + (TILE - 1)) >> TSHIFT
        incl = plsc.cumsum(padded)
        tile_start = incl - padded
        start_vec = tile_start * TILE + prefix_v

        base_a = wid * APW
        for v8 in range(APW // 16):
            vec = evmem[pl.ds(base_a + v8 * 16, 16)]
            pos = jnp.zeros((16,), jnp.int32)
            for e in range(NE):
                m = vec == e
                mi = m.astype(jnp.int32)
                cm = plsc.cumsum(mi)
                cnt = jnp.sum(mi)
                s_e = jnp.sum(jnp.where(lanes == e, start_vec, 0))
                pos = pos + jnp.where(m, s_e + cm - 1, 0)
                start_vec = start_vec + jnp.where(lanes == e, cnt, 0)
            posbuf[pl.ds(v8 * 16, 16)] = pos
            tokbuf[pl.ds(v8 * 16, 16)] = (base_a + v8 * 16 + lanes) >> 1

        pltpu.sync_copy(posbuf, pos_hbm.at[pl.ds(base_a, APW)])
        pltpu.async_copy(x_hbm.at[tokbuf], rows, sem).wait()
        pltpu.async_copy(rows, xs_hbm.at[posbuf], sem).wait()

        @pl.when(wid == 0)
        def _():
            ntiles = jnp.sum(jnp.where(lanes == NE - 1, incl, 0))
            for gv in range(3):
                g_vec = gv * 16 + lanes
                te = jnp.full((16,), -1, jnp.int32)
                for e in range(NE):
                    s_e = jnp.sum(jnp.where(lanes == e, tile_start, 0))
                    te = te + (g_vec >= s_e).astype(jnp.int32)
                metabuf[pl.ds(gv * 16, 16)] = te
            metabuf[pl.ds(48, 16)] = jnp.full((16,), ntiles, jnp.int32)
            pltpu.sync_copy(metabuf, meta_hbm)

    return k(e_flat, x_flat)


def _ffn_body(meta_ref, xs_ref, wg_ref, wu_ref, wd_ref, ys_ref):
    g = pl.program_id(0)
    ntiles = meta_ref[48]

    @pl.when(g < ntiles)
    def _():
        xt = xs_ref[...].astype(jnp.bfloat16)
        wg = wg_ref[0].astype(jnp.bfloat16)
        wu = wu_ref[0].astype(jnp.bfloat16)
        wd = wd_ref[0].astype(jnp.bfloat16)
        gg = jnp.dot(xt, wg, preferred_element_type=jnp.float32)
        s = gg * (1.0 / (1.0 + jnp.exp(-gg)))
        u = jnp.dot(xt, wu, preferred_element_type=jnp.float32)
        h = (s * u).astype(jnp.bfloat16)
        ys_ref[...] = jnp.dot(h, wd, preferred_element_type=jnp.float32)


def _ffn(meta, xs, Wg, Wu, Wd):
    grid_spec = pltpu.PrefetchScalarGridSpec(
        num_scalar_prefetch=1,
        grid=(GMAX,),
        in_specs=[
            pl.BlockSpec((TILE, EMBED), lambda g, te: (g, 0)),
            pl.BlockSpec((1, EMBED, FFN_D), lambda g, te: (te[g], 0, 0),
                         pipeline_mode=pl.Buffered(buffer_count=2)),
            pl.BlockSpec((1, EMBED, FFN_D), lambda g, te: (te[g], 0, 0),
                         pipeline_mode=pl.Buffered(buffer_count=2)),
            pl.BlockSpec((1, FFN_D, EMBED), lambda g, te: (te[g], 0, 0),
                         pipeline_mode=pl.Buffered(buffer_count=2)),
        ],
        out_specs=pl.BlockSpec((TILE, EMBED), lambda g, te: (g, 0)),
    )
    return pl.pallas_call(
        _ffn_body,
        grid_spec=grid_spec,
        out_shape=jax.ShapeDtypeStruct((SLOTS, EMBED), jnp.float32),
    )(meta, xs, Wg, Wu, Wd)


def _gather_back(ys, pos):
    mesh = plsc.VectorSubcoreMesh(core_axis_name="c", subcore_axis_name="s", num_cores=2, num_subcores=16)

    @functools.partial(
        pl.kernel,
        out_type=jax.ShapeDtypeStruct((NA, EMBED), jnp.float32),
        mesh=mesh,
        compiler_params=pltpu.CompilerParams(needs_layout_passes=False),
        scratch_types=[
            pltpu.VMEM((APW,), jnp.int32),
            pltpu.VMEM((APW,), jnp.int32),
            pltpu.VMEM((APW, EMBED), jnp.float32),
            pltpu.SemaphoreType.DMA,
        ],
    )
    def k(ys_hbm, pos_hbm, yg_hbm, posv, dstv, rows, sem):
        nc = 2
        wid = lax.axis_index("s") * nc + lax.axis_index("c")
        base = wid * APW
        lanes = lax.iota(jnp.int32, 16)
        pltpu.sync_copy(pos_hbm.at[pl.ds(base, APW)], posv)
        for v8 in range(APW // 16):
            a = base + v8 * 16 + lanes
            dstv[pl.ds(v8 * 16, 16)] = (a & 1) * NTOK + (a >> 1)
        pltpu.async_copy(ys_hbm.at[posv], rows, sem).wait()
        pltpu.async_copy(rows, yg_hbm.at[dstv], sem).wait()

    return k(ys, pos)


def _combine_body(w_ref, ya_ref, yb_ref, out_ref):
    m = pl.program_id(0)
    wm = w_ref[pl.ds(m * 256, 256), :]
    out_ref[...] = ya_ref[...] * wm[:, 0:1] + yb_ref[...] * wm[:, 1:2]


def _combine(w, yg):
    nblk = NTOK // 256
    return pl.pallas_call(
        _combine_body,
        grid=(nblk,),
        in_specs=[
            pl.BlockSpec((NTOK, 2), lambda m: (0, 0)),
            pl.BlockSpec((256, EMBED), lambda m: (m, 0)),
            pl.BlockSpec((256, EMBED), lambda m: (m + nblk, 0)),
        ],
        out_specs=pl.BlockSpec((256, EMBED), lambda m: (m, 0)),
        out_shape=jax.ShapeDtypeStruct((NTOK, EMBED), jnp.float32),
    )(w, yg, yg)


def kernel(x, Wr, Wg, Wu, Wd):
    B, T, D = x.shape
    x_flat = x.reshape(B * T, D)
    idx, w, loss = _router(x_flat, Wr)
    e_flat = idx.reshape(NA)
    pos, xs, meta = _dispatch(e_flat, x_flat)
    ys = _ffn(meta, xs, Wg, Wu, Wd)
    yg = _gather_back(ys, pos)
    out_flat = _combine(w, yg)
    return out_flat.reshape(B, T, D), loss.reshape(())
